# baseline (speedup 1.0000x reference)
import jax
import jax.numpy as jnp
from jax import lax
from jax.experimental import pallas as pl
from jax.experimental.pallas import tpu as pltpu
from jax.experimental.pallas import tpu_sc as plsc

ENTITY_DIM2 = 256
RELATION_DIM = 128
IN_DIM = ENTITY_DIM2 + RELATION_DIM
HIDDEN = 512
BATCH = 4096

_info = plsc.get_sparse_core_info()
_NC, _NS = _info.num_cores, _info.num_subcores
_NW = _NC * _NS
_BPW = BATCH // _NW
_HALF = _BPW // 2


def _gather_body(eids_hbm, pids_hbm, etab_hbm, rtab_hbm, emb_hbm, rel_hbm,
                 eidx_v, erows_v, pidx_v, prows_v, gsem_e0, gsem_e1, gsem_r,
                 wsem_e0, wsem_e1, wsem_r):
    wid = lax.axis_index("s") * _NC + lax.axis_index("c")
    base = wid * _BPW
    pltpu.sync_copy(eids_hbm.at[pl.ds(base, _BPW)], eidx_v)
    pltpu.sync_copy(pids_hbm.at[pl.ds(base, _BPW)], pidx_v)
    ge0 = pltpu.async_copy(etab_hbm.at[eidx_v.at[pl.ds(0, _HALF)]],
                           erows_v.at[pl.ds(0, _HALF)], gsem_e0)
    ge1 = pltpu.async_copy(etab_hbm.at[eidx_v.at[pl.ds(_HALF, _HALF)]],
                           erows_v.at[pl.ds(_HALF, _HALF)], gsem_e1)
    gr = pltpu.async_copy(rtab_hbm.at[pidx_v], prows_v, gsem_r)
    ge0.wait()
    we0 = pltpu.async_copy(erows_v.at[pl.ds(0, _HALF)],
                           emb_hbm.at[pl.ds(base, _HALF)], wsem_e0)
    gr.wait()
    wr = pltpu.async_copy(prows_v, rel_hbm.at[pl.ds(base, _BPW)], wsem_r)
    ge1.wait()
    we1 = pltpu.async_copy(erows_v.at[pl.ds(_HALF, _HALF)],
                           emb_hbm.at[pl.ds(base + _HALF, _HALF)], wsem_e1)
    we0.wait()
    wr.wait()
    we1.wait()


_sc_gather = pl.kernel(
    _gather_body,
    out_type=(
        jax.ShapeDtypeStruct((BATCH, ENTITY_DIM2), jnp.float32),
        jax.ShapeDtypeStruct((BATCH, RELATION_DIM), jnp.float32),
    ),
    mesh=plsc.VectorSubcoreMesh(core_axis_name="c", subcore_axis_name="s"),
    scratch_types=[
        pltpu.VMEM((_BPW,), jnp.int32),
        pltpu.VMEM((_BPW, ENTITY_DIM2), jnp.float32),
        pltpu.VMEM((_BPW,), jnp.int32),
        pltpu.VMEM((_BPW, RELATION_DIM), jnp.float32),
        pltpu.SemaphoreType.DMA,
        pltpu.SemaphoreType.DMA,
        pltpu.SemaphoreType.DMA,
        pltpu.SemaphoreType.DMA,
        pltpu.SemaphoreType.DMA,
        pltpu.SemaphoreType.DMA,
    ],
)

_BM = 512


def _mlp_body(emb_ref, rel_ref, W1_ref, b1_ref, W2_ref, b2_ref, W0_ref,
              b0_ref, out_ref):
    bf = jnp.bfloat16
    mm = lambda a, b: jnp.dot(a, b, preferred_element_type=jnp.float32)
    e = jnp.clip(emb_ref[...] + 1.0, 0.05, 1.0e9).astype(bf)
    r = rel_ref[...].astype(bf)
    W1 = W1_ref[...].astype(bf)
    h = (mm(e, W1[:ENTITY_DIM2]) + mm(r, W1[ENTITY_DIM2:])
         + b1_ref[...][None, :])
    h = jnp.maximum(h, 0.0).astype(bf)
    h = mm(h, W2_ref[...].astype(bf)) + b2_ref[...][None, :]
    h = jnp.maximum(h, 0.0).astype(bf)
    o = mm(h, W0_ref[...].astype(bf)) + b0_ref[...][None, :]
    out_ref[...] = jnp.clip(o + 1.0, 0.05, 1.0e9)


def kernel(entity_ids, proj_ids, entity_table, relation_table,
           W1, b1, W2, b2, W0, b0):
    emb, rel = _sc_gather(entity_ids, proj_ids, entity_table, relation_table)
    bf = jnp.bfloat16
    return pl.pallas_call(
        _mlp_body,
        grid=(BATCH // _BM,),
        in_specs=[
            pl.BlockSpec((_BM, ENTITY_DIM2), lambda i: (i, 0)),
            pl.BlockSpec((_BM, RELATION_DIM), lambda i: (i, 0)),
            pl.BlockSpec((IN_DIM, HIDDEN), lambda i: (0, 0)),
            pl.BlockSpec((HIDDEN,), lambda i: (0,)),
            pl.BlockSpec((HIDDEN, HIDDEN), lambda i: (0, 0)),
            pl.BlockSpec((HIDDEN,), lambda i: (0,)),
            pl.BlockSpec((HIDDEN, ENTITY_DIM2), lambda i: (0, 0)),
            pl.BlockSpec((ENTITY_DIM2,), lambda i: (0,)),
        ],
        out_specs=pl.BlockSpec((_BM, ENTITY_DIM2), lambda i: (i, 0)),
        out_shape=jax.ShapeDtypeStruct((BATCH, ENTITY_DIM2), jnp.float32),
    )(emb, rel, W1, b1, W2, b2, W0, b0)


# speedup vs baseline: 1.0582x; 1.0582x over previous
import jax
import jax.numpy as jnp
from jax import lax
from jax.experimental import pallas as pl
from jax.experimental.pallas import tpu as pltpu
from jax.experimental.pallas import tpu_sc as plsc

ENTITY_DIM2 = 256
RELATION_DIM = 128
IN_DIM = ENTITY_DIM2 + RELATION_DIM
HIDDEN = 512
BATCH = 4096

_info = plsc.get_sparse_core_info()
_NC, _NS = _info.num_cores, _info.num_subcores
_NW = _NC * _NS
_BPW = BATCH // _NW
_HALF = _BPW // 2


def _gather_body(eids_hbm, pids_hbm, etab_hbm, rtab_hbm, emb_hbm, rel_hbm,
                 eidx_v, erows_v, pidx_v, prows_v, gsem_e0, gsem_e1, gsem_r,
                 wsem_e0, wsem_e1, wsem_r):
    wid = lax.axis_index("s") * _NC + lax.axis_index("c")
    base = wid * _BPW
    pltpu.sync_copy(eids_hbm.at[pl.ds(base, _BPW)], eidx_v)
    pltpu.sync_copy(pids_hbm.at[pl.ds(base, _BPW)], pidx_v)
    ge0 = pltpu.async_copy(etab_hbm.at[eidx_v.at[pl.ds(0, _HALF)]],
                           erows_v.at[pl.ds(0, _HALF)], gsem_e0)
    ge1 = pltpu.async_copy(etab_hbm.at[eidx_v.at[pl.ds(_HALF, _HALF)]],
                           erows_v.at[pl.ds(_HALF, _HALF)], gsem_e1)
    gr = pltpu.async_copy(rtab_hbm.at[pidx_v], prows_v, gsem_r)
    ge0.wait()
    we0 = pltpu.async_copy(erows_v.at[pl.ds(0, _HALF)],
                           emb_hbm.at[pl.ds(base, _HALF)], wsem_e0)
    gr.wait()
    wr = pltpu.async_copy(prows_v, rel_hbm.at[pl.ds(base, _BPW)], wsem_r)
    ge1.wait()
    we1 = pltpu.async_copy(erows_v.at[pl.ds(_HALF, _HALF)],
                           emb_hbm.at[pl.ds(base + _HALF, _HALF)], wsem_e1)
    we0.wait()
    wr.wait()
    we1.wait()


_sc_gather = pl.kernel(
    _gather_body,
    out_type=(
        jax.ShapeDtypeStruct((BATCH, ENTITY_DIM2), jnp.float32),
        jax.ShapeDtypeStruct((BATCH, RELATION_DIM), jnp.float32),
    ),
    mesh=plsc.VectorSubcoreMesh(core_axis_name="c", subcore_axis_name="s"),
    scratch_types=[
        pltpu.VMEM((_BPW,), jnp.int32),
        pltpu.VMEM((_BPW, ENTITY_DIM2), jnp.float32),
        pltpu.VMEM((_BPW,), jnp.int32),
        pltpu.VMEM((_BPW, RELATION_DIM), jnp.float32),
        pltpu.SemaphoreType.DMA,
        pltpu.SemaphoreType.DMA,
        pltpu.SemaphoreType.DMA,
        pltpu.SemaphoreType.DMA,
        pltpu.SemaphoreType.DMA,
        pltpu.SemaphoreType.DMA,
    ],
)

_BM = 2048


def _mlp_body(emb_ref, rel_ref, W1_ref, b1_ref, W2_ref, b2_ref, W0_ref,
              b0_ref, out_ref):
    bf = jnp.bfloat16
    mm = lambda a, b: jnp.dot(a, b, preferred_element_type=jnp.float32)
    e = jnp.clip(emb_ref[...] + 1.0, 0.05, 1.0e9).astype(bf)
    r = rel_ref[...].astype(bf)
    W1 = W1_ref[...].astype(bf)
    h = (mm(e, W1[:ENTITY_DIM2]) + mm(r, W1[ENTITY_DIM2:])
         + b1_ref[...][None, :])
    h = jnp.maximum(h, 0.0).astype(bf)
    h = mm(h, W2_ref[...].astype(bf)) + b2_ref[...][None, :]
    h = jnp.maximum(h, 0.0).astype(bf)
    o = mm(h, W0_ref[...].astype(bf)) + b0_ref[...][None, :]
    out_ref[...] = jnp.clip(o + 1.0, 0.05, 1.0e9)


def kernel(entity_ids, proj_ids, entity_table, relation_table,
           W1, b1, W2, b2, W0, b0):
    emb, rel = _sc_gather(entity_ids, proj_ids, entity_table, relation_table)
    bf = jnp.bfloat16
    return pl.pallas_call(
        _mlp_body,
        grid=(BATCH // _BM,),
        in_specs=[
            pl.BlockSpec((_BM, ENTITY_DIM2), lambda i: (i, 0)),
            pl.BlockSpec((_BM, RELATION_DIM), lambda i: (i, 0)),
            pl.BlockSpec((IN_DIM, HIDDEN), lambda i: (0, 0)),
            pl.BlockSpec((HIDDEN,), lambda i: (0,)),
            pl.BlockSpec((HIDDEN, HIDDEN), lambda i: (0, 0)),
            pl.BlockSpec((HIDDEN,), lambda i: (0,)),
            pl.BlockSpec((HIDDEN, ENTITY_DIM2), lambda i: (0, 0)),
            pl.BlockSpec((ENTITY_DIM2,), lambda i: (0,)),
        ],
        out_specs=pl.BlockSpec((_BM, ENTITY_DIM2), lambda i: (i, 0)),
        out_shape=jax.ShapeDtypeStruct((BATCH, ENTITY_DIM2), jnp.float32),
    )(emb, rel, W1, b1, W2, b2, W0, b0)


# speedup vs baseline: 1.0589x; 1.0007x over previous
import jax
import jax.numpy as jnp
from jax import lax
from jax.experimental import pallas as pl
from jax.experimental.pallas import tpu as pltpu
from jax.experimental.pallas import tpu_sc as plsc

ENTITY_DIM2 = 256
RELATION_DIM = 128
IN_DIM = ENTITY_DIM2 + RELATION_DIM
HIDDEN = 512
BATCH = 4096

_info = plsc.get_sparse_core_info()
_NC, _NS = _info.num_cores, _info.num_subcores
_NW = _NC * _NS
_BPW = BATCH // _NW
_HALF = _BPW // 2


def _gather_body(eids_hbm, pids_hbm, etab_hbm, rtab_hbm, emb_hbm, rel_hbm,
                 eidx_v, erows_v, pidx_v, prows_v, gsem_e0, gsem_e1, gsem_r,
                 wsem_e0, wsem_e1, wsem_r):
    wid = lax.axis_index("s") * _NC + lax.axis_index("c")
    base = wid * _BPW
    pltpu.sync_copy(eids_hbm.at[pl.ds(base, _BPW)], eidx_v)
    pltpu.sync_copy(pids_hbm.at[pl.ds(base, _BPW)], pidx_v)
    ge0 = pltpu.async_copy(etab_hbm.at[eidx_v.at[pl.ds(0, _HALF)]],
                           erows_v.at[pl.ds(0, _HALF)], gsem_e0)
    ge1 = pltpu.async_copy(etab_hbm.at[eidx_v.at[pl.ds(_HALF, _HALF)]],
                           erows_v.at[pl.ds(_HALF, _HALF)], gsem_e1)
    gr = pltpu.async_copy(rtab_hbm.at[pidx_v], prows_v, gsem_r)
    ge0.wait()
    we0 = pltpu.async_copy(erows_v.at[pl.ds(0, _HALF)],
                           emb_hbm.at[pl.ds(base, _HALF)], wsem_e0)
    gr.wait()
    wr = pltpu.async_copy(prows_v, rel_hbm.at[pl.ds(base, _BPW)], wsem_r)
    ge1.wait()
    we1 = pltpu.async_copy(erows_v.at[pl.ds(_HALF, _HALF)],
                           emb_hbm.at[pl.ds(base + _HALF, _HALF)], wsem_e1)
    we0.wait()
    wr.wait()
    we1.wait()


_sc_gather = pl.kernel(
    _gather_body,
    out_type=(
        jax.ShapeDtypeStruct((BATCH, ENTITY_DIM2), jnp.float32),
        jax.ShapeDtypeStruct((BATCH, RELATION_DIM), jnp.float32),
    ),
    mesh=plsc.VectorSubcoreMesh(core_axis_name="c", subcore_axis_name="s"),
    scratch_types=[
        pltpu.VMEM((_BPW,), jnp.int32),
        pltpu.VMEM((_BPW, ENTITY_DIM2), jnp.float32),
        pltpu.VMEM((_BPW,), jnp.int32),
        pltpu.VMEM((_BPW, RELATION_DIM), jnp.float32),
        pltpu.SemaphoreType.DMA,
        pltpu.SemaphoreType.DMA,
        pltpu.SemaphoreType.DMA,
        pltpu.SemaphoreType.DMA,
        pltpu.SemaphoreType.DMA,
        pltpu.SemaphoreType.DMA,
    ],
)

_BM = 2048


def _mlp_body(emb_ref, rel_ref, W1_ref, b1_ref, W2_ref, b2_ref, W0_ref,
              b0_ref, out_ref):
    bf = jnp.bfloat16
    mm = lambda a, b: jnp.dot(a, b, preferred_element_type=jnp.float32)
    e = jnp.clip(emb_ref[...] + 1.0, 0.05, 1.0e9).astype(bf)
    r = rel_ref[...].astype(bf)
    W1 = W1_ref[...]
    h = (mm(e, W1[:ENTITY_DIM2]) + mm(r, W1[ENTITY_DIM2:])
         + b1_ref[...][None, :])
    h = jnp.maximum(h, 0.0).astype(bf)
    h = mm(h, W2_ref[...]) + b2_ref[...][None, :]
    h = jnp.maximum(h, 0.0).astype(bf)
    o = mm(h, W0_ref[...]) + b0_ref[...][None, :]
    out_ref[...] = jnp.clip(o + 1.0, 0.05, 1.0e9)


def kernel(entity_ids, proj_ids, entity_table, relation_table,
           W1, b1, W2, b2, W0, b0):
    emb, rel = _sc_gather(entity_ids, proj_ids, entity_table, relation_table)
    bf = jnp.bfloat16
    return pl.pallas_call(
        _mlp_body,
        grid=(BATCH // _BM,),
        in_specs=[
            pl.BlockSpec((_BM, ENTITY_DIM2), lambda i: (i, 0)),
            pl.BlockSpec((_BM, RELATION_DIM), lambda i: (i, 0)),
            pl.BlockSpec((IN_DIM, HIDDEN), lambda i: (0, 0)),
            pl.BlockSpec((HIDDEN,), lambda i: (0,)),
            pl.BlockSpec((HIDDEN, HIDDEN), lambda i: (0, 0)),
            pl.BlockSpec((HIDDEN,), lambda i: (0,)),
            pl.BlockSpec((HIDDEN, ENTITY_DIM2), lambda i: (0, 0)),
            pl.BlockSpec((ENTITY_DIM2,), lambda i: (0,)),
        ],
        out_specs=pl.BlockSpec((_BM, ENTITY_DIM2), lambda i: (i, 0)),
        out_shape=jax.ShapeDtypeStruct((BATCH, ENTITY_DIM2), jnp.float32),
    )(emb, rel, W1.astype(bf), b1, W2.astype(bf), b2, W0.astype(bf), b0)
